# baseline (speedup 1.0000x reference)
import jax
import jax.numpy as jnp
from jax import lax
from jax.experimental import pallas as pl
from jax.experimental.pallas import tpu as pltpu
from jax.experimental.pallas import tpu_sc as plsc

N_NODES = 10000
D_FEAT = 256
HALF = D_FEAT // 2
NUM_SC = 2
NUM_TEC = 16
CHUNK = 128
NROWS = 3
NIDX = 6
PERIOD = 6

ACC_ROWS = 10016
ZERO_ROWS = ACC_ROWS // NUM_TEC
OUT_ROWS = 624
TAIL_ROWS = N_NODES - NUM_TEC * OUT_ROWS


def _sc_kernel(e_pad, n_chunks):
    per_tile = n_chunks * CHUNK
    assert n_chunks % PERIOD == 0 and n_chunks > 2 * PERIOD

    def body(xs_hbm, src_hbm, dst_hbm, zer_hbm, out_hbm,
             src0, src1, src2, src3, src4, src5,
             dst0, dst1, dst2, dst3, dst4, dst5,
             rows0, rows1, rows2, acc,
             zsem, isem0, isem1, isem2, isem3, isem4, isem5,
             gsem0, gsem1, gsem2, ssem0, ssem1, ssem2):
        src_v = (src0, src1, src2, src3, src4, src5)
        dst_v = (dst0, dst1, dst2, dst3, dst4, dst5)
        rows = (rows0, rows1, rows2)
        isem = (isem0, isem1, isem2, isem3, isem4, isem5)
        gsem = (gsem0, gsem1, gsem2)
        ssem = (ssem0, ssem1, ssem2)
        c = lax.axis_index("c")
        s = lax.axis_index("s")
        base = c * e_pad + s * per_tile

        pltpu.async_copy(
            zer_hbm, acc.at[pl.ds(s * ZERO_ROWS, ZERO_ROWS)], zsem).wait()
        plsc.subcore_barrier()

        def istart(k, q):
            e0 = base + k * CHUNK
            pltpu.async_copy(src_hbm.at[pl.ds(e0, CHUNK)], src_v[q], isem[q])
            pltpu.async_copy(dst_hbm.at[pl.ds(e0, CHUNK)], dst_v[q], isem[q])

        def iwait(q):
            pltpu.make_async_copy(src_hbm.at[pl.ds(0, CHUNK)], src_v[q],
                                  isem[q]).wait()
            pltpu.make_async_copy(dst_hbm.at[pl.ds(0, CHUNK)], dst_v[q],
                                  isem[q]).wait()

        def gather_start(r, q):
            pltpu.async_copy(xs_hbm.at[src_v[q]], rows[r], gsem[r])

        def gather_wait(r, q):
            pltpu.make_async_copy(xs_hbm.at[src_v[q]], rows[r],
                                  gsem[r]).wait()

        def scatter_start(r, q):
            pltpu.async_copy(rows[r], acc.at[dst_v[q]], ssem[r], add=True)

        def scatter_wait(r, q):
            pltpu.make_async_copy(rows[r], acc.at[dst_v[q]],
                                  ssem[r]).wait()

        def slot(k, m):
            r0, q0 = m % NROWS, m % NIDX
            r1, q1 = (m + 1) % NROWS, (m + 1) % NIDX
            r2, q2 = (m - 2) % NROWS, (m - 2) % NIDX

            @pl.when(k >= 2)
            def _retire():
                scatter_wait(r2, q2)
            iwait(q1)
            gather_start(r1, q1)
            istart(k + 2, (m + 2) % NIDX)
            gather_wait(r0, q0)
            scatter_start(r0, q0)

        istart(0, 0)
        istart(1, 1)
        iwait(0)
        gather_start(0, 0)

        def outer(o, carry):
            k0 = o * PERIOD
            for m in range(PERIOD):
                slot(k0 + m, m)
            return carry

        lax.fori_loop(0, n_chunks // PERIOD, outer, 0)

        n = n_chunks
        gather_wait(n % NROWS, n % NIDX)
        iwait((n + 1) % NIDX)
        scatter_wait((n - 2) % NROWS, (n - 2) % NIDX)
        scatter_wait((n - 1) % NROWS, (n - 1) % NIDX)
        plsc.subcore_barrier()

        pltpu.sync_copy(
            acc.at[pl.ds(s * OUT_ROWS, OUT_ROWS)],
            out_hbm.at[pl.ds(s * OUT_ROWS, OUT_ROWS), pl.ds(c * HALF, HALF)])

        @pl.when(s == NUM_TEC - 1)
        def _tail():
            r0 = NUM_TEC * OUT_ROWS
            pltpu.sync_copy(
                acc.at[pl.ds(r0, TAIL_ROWS)],
                out_hbm.at[pl.ds(r0, TAIL_ROWS), pl.ds(c * HALF, HALF)])

    mesh = plsc.VectorSubcoreMesh(core_axis_name="c", subcore_axis_name="s")
    return pl.kernel(
        body,
        out_type=jax.ShapeDtypeStruct((N_NODES, D_FEAT), jnp.float32),
        mesh=mesh,
        scratch_types=(
            [pltpu.VMEM((CHUNK,), jnp.int32)] * (2 * NIDX)
            + [pltpu.VMEM((CHUNK, HALF), jnp.float32)] * NROWS
            + [pltpu.VMEM_SHARED((ACC_ROWS, HALF), jnp.float32)]
            + [pltpu.SemaphoreType.DMA] * (1 + NIDX + 2 * NROWS)
        ),
    )


@jax.jit
def kernel(x, up_index, down_index):
    n_edges = up_index.shape[1] + down_index.shape[1]
    align = NUM_TEC * CHUNK * PERIOD
    e_pad = ((n_edges + align - 1) // align) * align
    n_chunks = e_pad // (NUM_TEC * CHUNK)
    pad = e_pad - n_edges

    src = jnp.concatenate(
        [up_index[0], down_index[0], jnp.zeros((pad,), up_index.dtype)]
    ).astype(jnp.int32)
    dst = jnp.concatenate(
        [up_index[1], down_index[1],
         jnp.full((pad,), N_NODES, up_index.dtype)]
    ).astype(jnp.int32)
    extra_s = jnp.zeros((2 * CHUNK,), jnp.int32)
    extra_d = jnp.full((2 * CHUNK,), N_NODES, jnp.int32)
    src_all = jnp.concatenate([src, src + N_NODES, extra_s])
    dst_all = jnp.concatenate([dst, dst, extra_d])
    xs = jnp.concatenate([x[:, :HALF], x[:, HALF:]], axis=0)
    zer = jnp.zeros((ZERO_ROWS, HALF), jnp.float32)

    return _sc_kernel(e_pad, n_chunks)(xs, src_all, dst_all, zer)


# speedup vs baseline: 1.3593x; 1.3593x over previous
import jax
import jax.numpy as jnp
from jax import lax
from jax.experimental import pallas as pl
from jax.experimental.pallas import tpu as pltpu
from jax.experimental.pallas import tpu_sc as plsc

N_NODES = 10000
D_FEAT = 256
HALF = D_FEAT // 2
NUM_SC = 2
NUM_TEC = 16
CHUNK = 128

ACC_ROWS = 10016
ZERO_ROWS = ACC_ROWS // NUM_TEC
OUT_ROWS = 624
TAIL_ROWS = N_NODES - NUM_TEC * OUT_ROWS


def _sc_kernel(n_chunks):
    def body(xs_hbm, idx_hbm, zer_hbm, out_hbm,
             idx_v, rows_v, acc, zsem, gsem):
        c = lax.axis_index("c")
        s = lax.axis_index("s")
        ci0 = (c * NUM_TEC + s) * n_chunks

        pltpu.async_copy(
            zer_hbm, acc.at[pl.ds(s * ZERO_ROWS, ZERO_ROWS)], zsem).wait()
        plsc.subcore_barrier()

        def chunk(g, carry):
            pltpu.sync_copy(idx_hbm.at[ci0 + g], idx_v)
            pltpu.async_copy(xs_hbm.at[idx_v.at[0]], rows_v, gsem).wait()
            pltpu.sync_copy(rows_v, acc.at[idx_v.at[1]], add=True)
            return carry

        lax.fori_loop(0, n_chunks, chunk, 0)
        plsc.subcore_barrier()

        pltpu.sync_copy(
            acc.at[pl.ds(s * OUT_ROWS, OUT_ROWS)],
            out_hbm.at[pl.ds(s * OUT_ROWS, OUT_ROWS), pl.ds(c * HALF, HALF)])

        @pl.when(s == NUM_TEC - 1)
        def _tail():
            r0 = NUM_TEC * OUT_ROWS
            pltpu.sync_copy(
                acc.at[pl.ds(r0, TAIL_ROWS)],
                out_hbm.at[pl.ds(r0, TAIL_ROWS), pl.ds(c * HALF, HALF)])

    mesh = plsc.VectorSubcoreMesh(core_axis_name="c", subcore_axis_name="s")
    return pl.kernel(
        body,
        out_type=jax.ShapeDtypeStruct((N_NODES, D_FEAT), jnp.float32),
        mesh=mesh,
        scratch_types=[
            pltpu.VMEM((2, CHUNK), jnp.int32),
            pltpu.VMEM((CHUNK, HALF), jnp.float32),
            pltpu.VMEM_SHARED((ACC_ROWS, HALF), jnp.float32),
            pltpu.SemaphoreType.DMA,
            pltpu.SemaphoreType.DMA,
        ],
    )


@jax.jit
def kernel(x, up_index, down_index):
    n_edges = up_index.shape[1] + down_index.shape[1]
    align = NUM_TEC * CHUNK
    e_pad = ((n_edges + align - 1) // align) * align
    n_chunks = e_pad // align
    pad = e_pad - n_edges

    src = jnp.concatenate(
        [up_index[0], down_index[0], jnp.zeros((pad,), up_index.dtype)]
    ).astype(jnp.int32)
    dst = jnp.concatenate(
        [up_index[1], down_index[1],
         jnp.full((pad,), N_NODES, up_index.dtype)]
    ).astype(jnp.int32)
    src_all = jnp.concatenate([src, src + N_NODES]).reshape(-1, 1, CHUNK)
    dst_all = jnp.concatenate([dst, dst]).reshape(-1, 1, CHUNK)
    idx_all = jnp.concatenate([src_all, dst_all], axis=1)
    xs = jnp.concatenate([x[:, :HALF], x[:, HALF:]], axis=0)
    zer = jnp.zeros((ZERO_ROWS, HALF), jnp.float32)

    return _sc_kernel(n_chunks)(xs, idx_all, zer)
